# baseline (speedup 1.0000x reference)
import functools

import jax
import jax.numpy as jnp
from jax import lax
from jax.experimental import pallas as pl
from jax.experimental.pallas import tpu as pltpu
from jax.experimental.pallas import tpu_sc as plsc


def kernel(x, table):
    (B,) = x.shape
    V, D = table.shape
    info = plsc.get_sparse_core_info()
    NC, NS = info.num_cores, info.num_subcores
    NW = NC * NS
    b_per_w = B // NW

    mesh = plsc.VectorSubcoreMesh(core_axis_name="c", subcore_axis_name="s")

    @functools.partial(
        pl.kernel,
        mesh=mesh,
        out_type=jax.ShapeDtypeStruct((B, D), jnp.float32),
        scratch_types=[
            pltpu.VMEM((b_per_w,), jnp.int32),
            pltpu.VMEM((b_per_w, D), jnp.float32),
            pltpu.SemaphoreType.DMA,
        ],
        compiler_params=pltpu.CompilerParams(use_tc_tiling_on_sc=False),
    )
    def emb(idx_hbm, table_hbm, out_hbm, idx_v, rows_v, sem):
        wid = lax.axis_index("s") * NC + lax.axis_index("c")
        base = wid * b_per_w
        pltpu.sync_copy(idx_hbm.at[pl.ds(base, b_per_w)], idx_v)
        pltpu.async_copy(table_hbm.at[idx_v], rows_v, sem).wait()
        pltpu.sync_copy(rows_v, out_hbm.at[pl.ds(base, b_per_w)])

    return emb(x.astype(jnp.int32), table)


# speedup vs baseline: 3.8206x; 3.8206x over previous
import functools

import jax
import jax.numpy as jnp
from jax import lax
from jax.experimental import pallas as pl
from jax.experimental.pallas import tpu as pltpu
from jax.experimental.pallas import tpu_sc as plsc

_LANES = 16


def kernel(x, table):
    (B,) = x.shape
    V, D = table.shape
    info = plsc.get_sparse_core_info()
    NC, NS = info.num_cores, info.num_subcores
    NW = NC * NS
    b_per_w = B // NW
    WAVE = 8
    n_waves = b_per_w // WAVE

    tableT = table.T
    mesh = plsc.VectorSubcoreMesh(core_axis_name="c", subcore_axis_name="s")

    @functools.partial(
        pl.kernel,
        mesh=mesh,
        out_type=jax.ShapeDtypeStruct((D, B), jnp.float32),
        scratch_types=[
            pltpu.VMEM((b_per_w,), jnp.int32),
            pltpu.VMEM((2, WAVE, D, 128), jnp.float32),
            pltpu.VMEM((D, b_per_w), jnp.float32),
            pltpu.SemaphoreType.DMA((2,)),
        ],
        compiler_params=pltpu.CompilerParams(needs_layout_passes=False),
    )
    def emb(idx_hbm, tab_hbm, out_hbm, idx_v, dbuf, panel_v, sem):
        wid = lax.axis_index("s") * NC + lax.axis_index("c")
        base = wid * b_per_w
        pltpu.sync_copy(idx_hbm.at[pl.ds(base, b_per_w)], idx_v)

        def wave_idx(w, k):
            sel = plsc.load_gather(
                idx_v, [jnp.full((_LANES,), w * WAVE + k, jnp.int32)])
            return sel, sel[0]

        def fire_wave(w):
            def fire(k, _):
                _, r = wave_idx(w, k)
                p = pl.multiple_of(
                    lax.shift_right_logical(r, 7) * 128, 128)
                pltpu.async_copy(tab_hbm.at[:, pl.ds(p, 128)],
                                 dbuf.at[lax.rem(w, 2), k],
                                 sem.at[lax.rem(w, 2)])
                return 0
            lax.fori_loop(0, WAVE, fire, 0)

        def drain_wave(w):
            pltpu.make_async_copy(tab_hbm.at[:, pl.ds(0, WAVE * 128)],
                                  dbuf.at[lax.rem(w, 2)],
                                  sem.at[lax.rem(w, 2)]).wait()

        fire_wave(0)

        def step(w, _):
            @pl.when(w + 1 < n_waves)
            def _():
                fire_wave(w + 1)

            drain_wave(w)

            def extract(k, _):
                rvec, r = wave_idx(w, k)
                lane = lax.bitwise_and(rvec, 127)
                kk = jnp.full((_LANES,), k, jnp.int32)
                ww = jnp.full((_LANES,), lax.rem(w, 2), jnp.int32)
                ii = jnp.full((_LANES,), w * WAVE + k, jnp.int32)
                for c0 in range(0, D, _LANES):
                    cs = lax.iota(jnp.int32, _LANES) + c0
                    vals = plsc.load_gather(dbuf, [ww, kk, cs, lane])
                    plsc.store_scatter(panel_v, [cs, ii], vals)
                return 0

            lax.fori_loop(0, WAVE, extract, 0)
            return 0

        lax.fori_loop(0, n_waves, step, 0)
        pltpu.sync_copy(panel_v, out_hbm.at[:, pl.ds(base, b_per_w)])

    outT = emb(x.astype(jnp.int32), tableT)
    return outT.T
